# baseline (speedup 1.0000x reference)
import functools

import jax
import jax.numpy as jnp
from jax import lax
from jax.experimental import pallas as pl
from jax.experimental.pallas import tpu as pltpu
from jax.experimental.pallas import tpu_sc as plsc

NC = 2
NS = 16
NW = NC * NS
CH = 128
CHG = 128
RG = 2
RH = 2


def _hist_body(e, seg, edges, zeros_h, out, sidx, didx, hist_s, hist_d, red,
               obuf, stage_s, stage_d):
    cid = lax.axis_index("c")
    sid = lax.axis_index("s")
    wid = sid * NC + cid
    npad = NS * seg
    ept = e // NW
    base = wid * ept

    pltpu.sync_copy(edges.at[pl.ds(base, ept)], sidx)
    pltpu.sync_copy(edges.at[pl.ds(e + base, ept)], didx)
    pltpu.sync_copy(zeros_h, hist_s)
    pltpu.sync_copy(zeros_h, hist_d)
    ones = jnp.full((16,), 1.0, jnp.float32)

    def body(i, carry):
        plsc.addupdate_scatter(hist_s, [sidx[pl.ds(i * 16, 16)]], ones)
        plsc.addupdate_scatter(hist_d, [didx[pl.ds(i * 16, 16)]], ones)
        return carry

    lax.fori_loop(0, ept // 16, body, 0)
    pltpu.sync_copy(hist_s, stage_s.at[sid])
    pltpu.sync_copy(hist_d, stage_d.at[sid])
    plsc.subcore_barrier()

    for which, stage in ((0, stage_s), (1, stage_d)):
        pltpu.sync_copy(stage.at[:, pl.ds(sid * seg, seg)], red)

        def rbody(c, carry):
            v = red[0, pl.ds(c * 16, 16)]
            for r in range(1, NS):
                v = v + red[r, pl.ds(c * 16, 16)]
            obuf[pl.ds(c * 16, 16)] = v
            return carry

        lax.fori_loop(0, seg // 16, rbody, 0)
        pltpu.sync_copy(
            obuf, out.at[pl.ds((cid * 2 + which) * npad + sid * seg, seg)])


def _agg_body(e, e2, cpt, rpt, h, edges, padi, zeros_a, out, sidx, didx, rows,
              acc, gsem, is0, is1, id0, id1):
    isem = ((is0, id0), (is1, id1))
    cid = lax.axis_index("c")
    sid = lax.axis_index("s")
    wid = sid * NC + cid

    pltpu.sync_copy(zeros_a, acc.at[pl.ds(sid * rpt, rpt)])
    plsc.subcore_barrier()

    def fire_idx(k, b):
        base = (wid + NW * k) * CHG

        @pl.when(base < e)
        def _():
            pltpu.async_copy(edges.at[pl.ds(base, CHG)], sidx.at[b],
                             isem[b][0])
            for j in range(CHG // CH):
                pltpu.async_copy(edges.at[pl.ds(e + base + j * CH, CH)],
                                 didx.at[b, j], isem[b][1])

        @pl.when(base >= e)
        def _():
            pltpu.async_copy(padi.at[pl.ds(base - e, CHG)], sidx.at[b],
                             isem[b][0])
            for j in range(CHG // CH):
                pltpu.async_copy(padi.at[pl.ds(base - e + j * CH, CH)],
                                 didx.at[b, j], isem[b][1])

    def absorb(b):
        pltpu.make_async_copy(edges.at[pl.ds(0, CHG)], sidx.at[b],
                              isem[b][0]).wait()
        for j in range(CHG // CH):
            pltpu.make_async_copy(edges.at[pl.ds(0, CH)], didx.at[b, j],
                                  isem[b][1]).wait()
        pltpu.async_copy(h.at[sidx.at[b]], rows, gsem).wait()
        for j in range(CHG // CH):
            pltpu.sync_copy(rows.at[pl.ds(j * CH, CH)],
                            acc.at[didx.at[b, j]], add=True)

    for b in range(RG):
        fire_idx(b, b)

    def group(g, carry):
        for b in range(RG):
            absorb(b)
            fire_idx(g * RG + b + RG, b)
        return carry

    lax.fori_loop(0, cpt // RG - 1, group, 0)
    for b in range(RG):
        absorb(b)

    plsc.subcore_barrier()
    pltpu.sync_copy(acc.at[pl.ds(sid * rpt, rpt)],
                    out.at[cid, pl.ds(sid * rpt, rpt)])


def _mm_body(x_ref, w_ref, o_ref):
    o_ref[...] = jnp.dot(x_ref[...], w_ref[...],
                         preferred_element_type=jnp.float32)


def _scale_body(deg_ref, g_ref, o_ref):
    norm = lax.rsqrt(jnp.maximum(deg_ref[...], 1.0))
    o_ref[...] = g_ref[...] * norm


def _fin_body(p_ref, deg_ref, b_ref, x_ref, o_ref):
    agg = p_ref[0] + p_ref[1]
    norm = lax.rsqrt(jnp.maximum(deg_ref[...], 1.0))
    o = jnp.maximum(agg * norm + b_ref[...], 0.0)
    o_ref[...] = o + x_ref[...]


def kernel(feats, edge_index, W, b):
    n, d = feats.shape
    e = edge_index.shape[1]
    lcm = RG * RH
    cpt = ((e + CH * NW - 1) // (CH * NW) + lcm - 1) // lcm * lcm
    e2 = NW * cpt * CH
    seg = ((n + NS * 128 - 1) // (NS * 128)) * 128
    npad = NS * seg
    rpt = seg
    npr = NS * rpt

    mesh = plsc.VectorSubcoreMesh(core_axis_name="c", subcore_axis_name="s")
    assert e % (NW * 16) == 0
    edge_flat = edge_index.reshape(2 * e)
    padi = n + (jnp.arange(max(e2 - e, CHG), dtype=jnp.int32)
                % jnp.int32(npr - n))

    hist = pl.kernel(
        functools.partial(_hist_body, e, seg),
        mesh=mesh,
        compiler_params=pltpu.CompilerParams(needs_layout_passes=False),
        out_type=jax.ShapeDtypeStruct((NC * 2 * npad,), jnp.float32),
        scratch_types=[
            pltpu.VMEM((e // NW,), jnp.int32),
            pltpu.VMEM((e // NW,), jnp.int32),
            pltpu.VMEM((npad,), jnp.float32),
            pltpu.VMEM((npad,), jnp.float32),
            pltpu.VMEM((NS, seg), jnp.float32),
            pltpu.VMEM((seg,), jnp.float32),
            pltpu.VMEM_SHARED((NS, npad), jnp.float32),
            pltpu.VMEM_SHARED((NS, npad), jnp.float32),
        ],
    )(edge_flat, jnp.zeros((npad,), jnp.float32))
    hist = hist.reshape(NC, 2, npad)

    deg_src = (hist[0, 0, :n] + hist[1, 0, :n]).reshape(n, 1)
    deg_dst = (hist[0, 1, :n] + hist[1, 1, :n]).reshape(n, 1)

    bm = 2000
    assert n % bm == 0
    g = pl.pallas_call(
        _mm_body,
        grid=(n // bm,),
        in_specs=[
            pl.BlockSpec((bm, d), lambda i: (i, 0)),
            pl.BlockSpec((d, d), lambda i: (0, 0)),
        ],
        out_specs=pl.BlockSpec((bm, d), lambda i: (i, 0)),
        out_shape=jax.ShapeDtypeStruct((n, d), jnp.float32),
    )(feats, W)
    h = pl.pallas_call(
        _scale_body,
        grid=(n // bm,),
        in_specs=[
            pl.BlockSpec((bm, 1), lambda i: (i, 0)),
            pl.BlockSpec((bm, d), lambda i: (i, 0)),
        ],
        out_specs=pl.BlockSpec((bm, d), lambda i: (i, 0)),
        out_shape=jax.ShapeDtypeStruct((npr, d), jnp.float32),
    )(deg_src, g)

    cpt_g = cpt * CH // CHG
    assert cpt_g % RG == 0 and e % CHG == 0
    parts = pl.kernel(
        functools.partial(_agg_body, e, e2, cpt_g, rpt),
        mesh=mesh,
        out_type=jax.ShapeDtypeStruct((NC, npr, d), jnp.float32),
        scratch_types=[
            pltpu.VMEM((RG, CHG), jnp.int32),
            pltpu.VMEM((RG, CHG // CH, CH), jnp.int32),
            pltpu.VMEM((CHG, d), jnp.float32),
            pltpu.VMEM_SHARED((npr, d), jnp.float32),
        ] + [pltpu.SemaphoreType.DMA] * 5,
    )(h, edge_flat, padi, jnp.zeros((rpt, d), jnp.float32))

    out = pl.pallas_call(
        _fin_body,
        grid=(n // bm,),
        in_specs=[
            pl.BlockSpec((NC, bm, d), lambda i: (0, i, 0)),
            pl.BlockSpec((bm, 1), lambda i: (i, 0)),
            pl.BlockSpec((1, d), lambda i: (0, 0)),
            pl.BlockSpec((bm, d), lambda i: (i, 0)),
        ],
        out_specs=pl.BlockSpec((bm, d), lambda i: (i, 0)),
        out_shape=jax.ShapeDtypeStruct((n, d), jnp.float32),
    )(parts, deg_dst, b.reshape(1, d), feats)

    return out


# speedup vs baseline: 1.0184x; 1.0184x over previous
import functools

import jax
import jax.numpy as jnp
from jax import lax
from jax.experimental import pallas as pl
from jax.experimental.pallas import tpu as pltpu
from jax.experimental.pallas import tpu_sc as plsc

NC = 2
NS = 16
NW = NC * NS
CH = 128
CHG = 128
RG = 2
RH = 2


def _hist_body(e, seg, edges, zeros_h, out, sidx, didx, hist_s, hist_d, red,
               obuf, stage_s, stage_d):
    cid = lax.axis_index("c")
    sid = lax.axis_index("s")
    wid = sid * NC + cid
    npad = NS * seg
    ept = e // NW
    base = wid * ept

    pltpu.sync_copy(edges.at[pl.ds(base, ept)], sidx)
    pltpu.sync_copy(edges.at[pl.ds(e + base, ept)], didx)
    pltpu.sync_copy(zeros_h, hist_s)
    pltpu.sync_copy(zeros_h, hist_d)
    ones = jnp.full((16,), 1.0, jnp.float32)

    def body(i, carry):
        plsc.addupdate_scatter(hist_s, [sidx[pl.ds(i * 16, 16)]], ones)
        plsc.addupdate_scatter(hist_d, [didx[pl.ds(i * 16, 16)]], ones)
        return carry

    lax.fori_loop(0, ept // 16, body, 0)
    pltpu.sync_copy(hist_s, stage_s.at[sid])
    pltpu.sync_copy(hist_d, stage_d.at[sid])
    plsc.subcore_barrier()

    for which, stage in ((0, stage_s), (1, stage_d)):
        pltpu.sync_copy(stage.at[:, pl.ds(sid * seg, seg)], red)

        def rbody(c, carry):
            v = red[0, pl.ds(c * 16, 16)]
            for r in range(1, NS):
                v = v + red[r, pl.ds(c * 16, 16)]
            obuf[pl.ds(c * 16, 16)] = v
            return carry

        lax.fori_loop(0, seg // 16, rbody, 0)
        pltpu.sync_copy(
            obuf, out.at[pl.ds((cid * 2 + which) * npad + sid * seg, seg)])


def _agg_body(e, cpt, rpt, h, edges, zeros_a, out, sidx, didx, rows,
              acc, gsem, is0, is1, id0, id1):
    isem = ((is0, id0), (is1, id1))
    cid = lax.axis_index("c")
    sid = lax.axis_index("s")
    wid = sid * NC + cid

    pltpu.sync_copy(zeros_a, acc.at[pl.ds(sid * rpt, rpt)])
    plsc.subcore_barrier()

    def fire_idx(k, b):
        base = (wid + NW * k) * CHG

        @pl.when(base < e)
        def _():
            pltpu.async_copy(edges.at[pl.ds(base, CHG)], sidx.at[b],
                             isem[b][0])
            for j in range(CHG // CH):
                pltpu.async_copy(edges.at[pl.ds(e + base + j * CH, CH)],
                                 didx.at[b, j], isem[b][1])

    def absorb(k, b):
        base = (wid + NW * k) * CHG

        @pl.when(base < e)
        def _():
            pltpu.make_async_copy(edges.at[pl.ds(0, CHG)], sidx.at[b],
                                  isem[b][0]).wait()
            for j in range(CHG // CH):
                pltpu.make_async_copy(edges.at[pl.ds(0, CH)], didx.at[b, j],
                                      isem[b][1]).wait()
            pltpu.async_copy(h.at[sidx.at[b]], rows, gsem).wait()
            for j in range(CHG // CH):
                pltpu.sync_copy(rows.at[pl.ds(j * CH, CH)],
                                acc.at[didx.at[b, j]], add=True)

    for b in range(RG):
        fire_idx(b, b)

    def group(g, carry):
        for b in range(RG):
            absorb(g * RG + b, b)
            fire_idx(g * RG + b + RG, b)
        return carry

    lax.fori_loop(0, cpt // RG - 1, group, 0)
    for b in range(RG):
        absorb(cpt - RG + b, b)

    plsc.subcore_barrier()
    pltpu.sync_copy(acc.at[pl.ds(sid * rpt, rpt)],
                    out.at[cid, pl.ds(sid * rpt, rpt)])


def _mm_body(deg_ref, x_ref, w_ref, o_ref):
    norm = lax.rsqrt(jnp.maximum(deg_ref[...], 1.0))
    o_ref[...] = jnp.dot(x_ref[...] * norm, w_ref[...],
                         preferred_element_type=jnp.float32)


def _fin_body(p_ref, deg_ref, b_ref, x_ref, o_ref):
    agg = p_ref[0] + p_ref[1]
    norm = lax.rsqrt(jnp.maximum(deg_ref[...], 1.0))
    o = jnp.maximum(agg * norm + b_ref[...], 0.0)
    o_ref[...] = o + x_ref[...]


def kernel(feats, edge_index, W, b):
    n, d = feats.shape
    e = edge_index.shape[1]
    lcm = RG * RH
    cpt = ((e + CH * NW - 1) // (CH * NW) + lcm - 1) // lcm * lcm
    e2 = NW * cpt * CH
    seg = ((n + NS * 128 - 1) // (NS * 128)) * 128
    npad = NS * seg
    rpt = seg
    npr = NS * rpt

    mesh = plsc.VectorSubcoreMesh(core_axis_name="c", subcore_axis_name="s")
    assert e % (NW * 16) == 0
    edge_flat = edge_index.reshape(2 * e)

    hist = pl.kernel(
        functools.partial(_hist_body, e, seg),
        mesh=mesh,
        compiler_params=pltpu.CompilerParams(needs_layout_passes=False),
        out_type=jax.ShapeDtypeStruct((NC * 2 * npad,), jnp.float32),
        scratch_types=[
            pltpu.VMEM((e // NW,), jnp.int32),
            pltpu.VMEM((e // NW,), jnp.int32),
            pltpu.VMEM((npad,), jnp.float32),
            pltpu.VMEM((npad,), jnp.float32),
            pltpu.VMEM((NS, seg), jnp.float32),
            pltpu.VMEM((seg,), jnp.float32),
            pltpu.VMEM_SHARED((NS, npad), jnp.float32),
            pltpu.VMEM_SHARED((NS, npad), jnp.float32),
        ],
    )(edge_flat, jnp.zeros((npad,), jnp.float32))
    hist = hist.reshape(NC, 2, npad)

    deg_src = (hist[0, 0, :n] + hist[1, 0, :n]).reshape(n, 1)
    deg_dst = (hist[0, 1, :n] + hist[1, 1, :n]).reshape(n, 1)

    bm = 2000
    assert n % bm == 0
    h = pl.pallas_call(
        _mm_body,
        grid=(n // bm,),
        in_specs=[
            pl.BlockSpec((bm, 1), lambda i: (i, 0)),
            pl.BlockSpec((bm, d), lambda i: (i, 0)),
            pl.BlockSpec((d, d), lambda i: (0, 0)),
        ],
        out_specs=pl.BlockSpec((bm, d), lambda i: (i, 0)),
        out_shape=jax.ShapeDtypeStruct((npr, d), jnp.float32),
    )(deg_src, feats, W)

    cpt_g = cpt * CH // CHG
    assert cpt_g % RG == 0 and e % CHG == 0
    parts = pl.kernel(
        functools.partial(_agg_body, e, cpt_g, rpt),
        mesh=mesh,
        out_type=jax.ShapeDtypeStruct((NC, npr, d), jnp.float32),
        scratch_types=[
            pltpu.VMEM((RG, CHG), jnp.int32),
            pltpu.VMEM((RG, CHG // CH, CH), jnp.int32),
            pltpu.VMEM((CHG, d), jnp.float32),
            pltpu.VMEM_SHARED((npr, d), jnp.float32),
        ] + [pltpu.SemaphoreType.DMA] * 5,
    )(h, edge_flat, jnp.zeros((rpt, d), jnp.float32))

    out = pl.pallas_call(
        _fin_body,
        grid=(n // bm,),
        in_specs=[
            pl.BlockSpec((NC, bm, d), lambda i: (0, i, 0)),
            pl.BlockSpec((bm, 1), lambda i: (i, 0)),
            pl.BlockSpec((1, d), lambda i: (0, 0)),
            pl.BlockSpec((bm, d), lambda i: (i, 0)),
        ],
        out_specs=pl.BlockSpec((bm, d), lambda i: (i, 0)),
        out_shape=jax.ShapeDtypeStruct((n, d), jnp.float32),
    )(parts, deg_dst, b.reshape(1, d), feats)

    return out


# speedup vs baseline: 1.2973x; 1.2738x over previous
import functools

import jax
import jax.numpy as jnp
from jax import lax
from jax.experimental import pallas as pl
from jax.experimental.pallas import tpu as pltpu
from jax.experimental.pallas import tpu_sc as plsc

NC = 2
NS = 16
NW = NC * NS
CH = 128
CHG = 128
RG = 2
RH = 2


def _hist_body(e, seg, edges, zeros_h, out, sidx, didx, hist_s, hist_d, red,
               obuf, stage_s, stage_d):
    cid = lax.axis_index("c")
    sid = lax.axis_index("s")
    wid = sid * NC + cid
    npad = NS * seg
    ept = e // NW
    base = wid * ept

    pltpu.sync_copy(edges.at[pl.ds(base, ept)], sidx)
    pltpu.sync_copy(edges.at[pl.ds(e + base, ept)], didx)
    pltpu.sync_copy(zeros_h, hist_s)
    pltpu.sync_copy(zeros_h, hist_d)
    ones = jnp.full((16,), 1.0, jnp.float32)

    def body(i, carry):
        plsc.addupdate_scatter(hist_s, [sidx[pl.ds(i * 16, 16)]], ones)
        plsc.addupdate_scatter(hist_d, [didx[pl.ds(i * 16, 16)]], ones)
        return carry

    lax.fori_loop(0, ept // 16, body, 0)
    pltpu.sync_copy(hist_s, stage_s.at[sid])
    pltpu.sync_copy(hist_d, stage_d.at[sid])
    plsc.subcore_barrier()

    for which, stage in ((0, stage_s), (1, stage_d)):
        pltpu.sync_copy(stage.at[:, pl.ds(sid * seg, seg)], red)

        def rbody(c, carry):
            v = red[0, pl.ds(c * 16, 16)]
            for r in range(1, NS):
                v = v + red[r, pl.ds(c * 16, 16)]
            obuf[pl.ds(c * 16, 16)] = v
            return carry

        lax.fori_loop(0, seg // 16, rbody, 0)
        pltpu.sync_copy(
            obuf, out.at[pl.ds((cid * 2 + which) * npad + sid * seg, seg)])


def _agg_body(e, cpt, rpt, h, edges, zeros_a, out, sidx, didx, rows,
              acc, gsem0, gsem1, is0, is1, id0, id1):
    isem = ((is0, id0), (is1, id1))
    cid = lax.axis_index("c")
    sid = lax.axis_index("s")
    wid = sid * NC + cid

    pltpu.sync_copy(zeros_a, acc.at[pl.ds(sid * rpt, rpt)])
    plsc.subcore_barrier()

    gsem = (gsem0, gsem1)

    def fire_idx(k, b):
        base = (wid + NW * k) * CHG

        @pl.when(base < e)
        def _():
            pltpu.async_copy(edges.at[pl.ds(base, CHG)], sidx.at[b],
                             isem[b][0])
            for j in range(CHG // CH):
                pltpu.async_copy(edges.at[pl.ds(e + base + j * CH, CH)],
                                 didx.at[b, j], isem[b][1])

    def fire_gather(k, b):
        base = (wid + NW * k) * CHG

        @pl.when(base < e)
        def _():
            pltpu.make_async_copy(edges.at[pl.ds(0, CHG)], sidx.at[b],
                                  isem[b][0]).wait()
            for j in range(CHG // CH):
                pltpu.make_async_copy(edges.at[pl.ds(0, CH)], didx.at[b, j],
                                      isem[b][1]).wait()
            pltpu.async_copy(h.at[sidx.at[b]], rows.at[b], gsem[b])

    def scat(k, b):
        base = (wid + NW * k) * CHG

        @pl.when(base < e)
        def _():
            pltpu.make_async_copy(h.at[sidx.at[b]], rows.at[b],
                                  gsem[b]).wait()
            for j in range(CHG // CH):
                pltpu.sync_copy(rows.at[b, pl.ds(j * CH, CH)],
                                acc.at[didx.at[b, j]], add=True)

    fire_idx(0, 0)
    fire_idx(1, 1)
    fire_gather(0, 0)

    def group(g, carry):
        for b in range(RG):
            k = g * RG + b
            fire_gather(k + 1, (b + 1) % RG)
            scat(k, b)
            fire_idx(k + 2, b)
        return carry

    lax.fori_loop(0, cpt // RG, group, 0)

    plsc.subcore_barrier()
    pltpu.sync_copy(acc.at[pl.ds(sid * rpt, rpt)],
                    out.at[cid, pl.ds(sid * rpt, rpt)])


def _mm_body(deg_ref, x_ref, w_ref, o_ref):
    norm = lax.rsqrt(jnp.maximum(deg_ref[...], 1.0))
    o_ref[...] = jnp.dot(x_ref[...] * norm, w_ref[...],
                         preferred_element_type=jnp.float32)


def _fin_body(p_ref, deg_ref, b_ref, x_ref, o_ref):
    agg = p_ref[0] + p_ref[1]
    norm = lax.rsqrt(jnp.maximum(deg_ref[...], 1.0))
    o = jnp.maximum(agg * norm + b_ref[...], 0.0)
    o_ref[...] = o + x_ref[...]


def kernel(feats, edge_index, W, b):
    n, d = feats.shape
    e = edge_index.shape[1]
    lcm = RG * RH
    cpt = ((e + CH * NW - 1) // (CH * NW) + lcm - 1) // lcm * lcm
    e2 = NW * cpt * CH
    seg = ((n + NS * 128 - 1) // (NS * 128)) * 128
    npad = NS * seg
    rpt = seg
    npr = NS * rpt

    mesh = plsc.VectorSubcoreMesh(core_axis_name="c", subcore_axis_name="s")
    assert e % (NW * 16) == 0
    edge_flat = edge_index.reshape(2 * e)

    hist = pl.kernel(
        functools.partial(_hist_body, e, seg),
        mesh=mesh,
        compiler_params=pltpu.CompilerParams(needs_layout_passes=False),
        out_type=jax.ShapeDtypeStruct((NC * 2 * npad,), jnp.float32),
        scratch_types=[
            pltpu.VMEM((e // NW,), jnp.int32),
            pltpu.VMEM((e // NW,), jnp.int32),
            pltpu.VMEM((npad,), jnp.float32),
            pltpu.VMEM((npad,), jnp.float32),
            pltpu.VMEM((NS, seg), jnp.float32),
            pltpu.VMEM((seg,), jnp.float32),
            pltpu.VMEM_SHARED((NS, npad), jnp.float32),
            pltpu.VMEM_SHARED((NS, npad), jnp.float32),
        ],
    )(edge_flat, jnp.zeros((npad,), jnp.float32))
    hist = hist.reshape(NC, 2, npad)

    deg_src = (hist[0, 0, :n] + hist[1, 0, :n]).reshape(n, 1)
    deg_dst = (hist[0, 1, :n] + hist[1, 1, :n]).reshape(n, 1)

    bm = 2000
    assert n % bm == 0
    h = pl.pallas_call(
        _mm_body,
        grid=(n // bm,),
        in_specs=[
            pl.BlockSpec((bm, 1), lambda i: (i, 0)),
            pl.BlockSpec((bm, d), lambda i: (i, 0)),
            pl.BlockSpec((d, d), lambda i: (0, 0)),
        ],
        out_specs=pl.BlockSpec((bm, d), lambda i: (i, 0)),
        out_shape=jax.ShapeDtypeStruct((npr, d), jnp.float32),
    )(deg_src, feats, W)

    cpt_g = cpt * CH // CHG
    assert cpt_g % RG == 0 and e % CHG == 0
    parts = pl.kernel(
        functools.partial(_agg_body, e, cpt_g, rpt),
        mesh=mesh,
        out_type=jax.ShapeDtypeStruct((NC, npr, d), jnp.float32),
        scratch_types=[
            pltpu.VMEM((RG, CHG), jnp.int32),
            pltpu.VMEM((RG, CHG // CH, CH), jnp.int32),
            pltpu.VMEM((RG, CHG, d), jnp.float32),
            pltpu.VMEM_SHARED((npr, d), jnp.float32),
        ] + [pltpu.SemaphoreType.DMA] * 6,
    )(h, edge_flat, jnp.zeros((rpt, d), jnp.float32))

    out = pl.pallas_call(
        _fin_body,
        grid=(n // bm,),
        in_specs=[
            pl.BlockSpec((NC, bm, d), lambda i: (0, i, 0)),
            pl.BlockSpec((bm, 1), lambda i: (i, 0)),
            pl.BlockSpec((1, d), lambda i: (0, 0)),
            pl.BlockSpec((bm, d), lambda i: (i, 0)),
        ],
        out_specs=pl.BlockSpec((bm, d), lambda i: (i, 0)),
        out_shape=jax.ShapeDtypeStruct((n, d), jnp.float32),
    )(parts, deg_dst, b.reshape(1, d), feats)

    return out
